# baseline (speedup 1.0000x reference)
import functools

import jax
import jax.numpy as jnp
from jax import lax
from jax.experimental import pallas as pl
from jax.experimental.pallas import tpu as pltpu
from jax.experimental.pallas import tpu_sc as plsc

N_CODES = 8192
DIM = 32
N_TOK = 8192
TOK_BLK = 512
CHUNK = 4096
N_ELEMS = float(N_TOK * DIM)


def _argmin_body(x_ref, a_ref, w_ref, idx_ref, d_ref):
    x = (x_ref[...] * -2.0).astype(jnp.bfloat16).astype(jnp.float32)
    a = a_ref[...]
    col = lax.broadcasted_iota(jnp.int32, (TOK_BLK, CHUNK), 1).astype(jnp.float32)

    def step(k, carry):
        bd, bi = carry
        w = w_ref[pl.ds(k * CHUNK, CHUNK), :]
        mm = lax.dot_general(x, w, (((1,), (1,)), ((), ())),
                             preferred_element_type=jnp.float32)
        d = a + mm
        dmin = jnp.min(d, axis=1, keepdims=True)
        cand = jnp.where(d == dmin, col, jnp.float32(2**24))
        amin = jnp.min(cand, axis=1, keepdims=True).astype(jnp.int32) + k * CHUNK
        upd = dmin < bd
        return jnp.where(upd, dmin, bd), jnp.where(upd, amin, bi)

    half = N_CODES // 2 // CHUNK
    bd0 = jnp.full((TOK_BLK, 1), jnp.inf, jnp.float32)
    bi0 = jnp.zeros((TOK_BLK, 1), jnp.int32)
    bd, bi = lax.fori_loop(0, half, step, (bd0, bi0))
    bd = bd.astype(jnp.bfloat16).astype(jnp.float32)
    bd, bi = lax.fori_loop(half, 2 * half, step, (bd, bi))
    idx_ref[...] = bi
    d_ref[...] = bd


def _argmin_call(flat, a, weight):
    return pl.pallas_call(
        _argmin_body,
        grid=(N_TOK // TOK_BLK,),
        in_specs=[
            pl.BlockSpec((TOK_BLK, DIM), lambda i: (i, 0)),
            pl.BlockSpec((TOK_BLK, 1), lambda i: (i, 0)),
            pl.BlockSpec((N_CODES, DIM), lambda i: (0, 0)),
        ],
        out_specs=[
            pl.BlockSpec((TOK_BLK, 1), lambda i: (i, 0)),
            pl.BlockSpec((TOK_BLK, 1), lambda i: (i, 0)),
        ],
        out_shape=[
            jax.ShapeDtypeStruct((N_TOK, 1), jnp.int32),
            jax.ShapeDtypeStruct((N_TOK, 1), jnp.float32),
        ],
    )(flat, a, weight)


def _sc_gather_hist(idx2d, weight128, zeros, ones):
    mesh = plsc.VectorSubcoreMesh(core_axis_name="c", subcore_axis_name="s")
    nc, ns = mesh.num_cores, mesh.num_subcores
    nw = nc * ns
    rows_per_w = N_TOK // 128 // nw

    @functools.partial(
        pl.kernel,
        out_type=[
            jax.ShapeDtypeStruct((N_TOK // 128, 128, 128), jnp.float32),
            jax.ShapeDtypeStruct((nc, N_CODES), jnp.float32),
        ],
        mesh=mesh,
        scratch_types=[
            pltpu.VMEM((rows_per_w, 128), jnp.int32),
            pltpu.VMEM((rows_per_w, 128, 128), jnp.float32),
            pltpu.VMEM((128,), jnp.float32),
            pltpu.VMEM_SHARED((N_CODES,), jnp.float32),
            pltpu.SemaphoreType.DMA,
        ],
    )
    def k(idx_hbm, w_hbm, z_hbm, one_hbm, q_hbm, cnt_hbm,
          idx_v, rows_v, ones_v, shared, sem):
        cid = lax.axis_index("c")
        sid = lax.axis_index("s")
        wid = sid * nc + cid
        base = wid * rows_per_w

        pltpu.sync_copy(idx_hbm.at[pl.ds(base, rows_per_w)], idx_v)
        copies = [
            pltpu.async_copy(w_hbm.at[idx_v.at[j]], rows_v.at[j], sem)
            for j in range(rows_per_w)
        ]
        @pl.when(sid == 0)
        def _():
            pltpu.sync_copy(z_hbm, shared)
        pltpu.sync_copy(one_hbm, ones_v)
        for c in copies:
            c.wait()
        pltpu.sync_copy(rows_v, q_hbm.at[pl.ds(base, rows_per_w)])

        plsc.subcore_barrier()
        for j in range(rows_per_w):
            pltpu.sync_copy(ones_v, shared.at[idx_v.at[j]], add=True)
        plsc.subcore_barrier()

        @pl.when(sid == 0)
        def _():
            pltpu.sync_copy(shared, cnt_hbm.at[cid])

    return k(idx2d, weight128, zeros, ones)


def _scalars_body(d_ref, c_ref, loss_ref, perp_ref):
    sumd = jnp.sum(d_ref[...], keepdims=True)
    loss_ref[...] = sumd * (1.25 / N_ELEMS)
    p = jnp.sum(c_ref[...], axis=0, keepdims=True) * (1.0 / N_TOK)
    ent = -jnp.sum(p * jnp.log(p + 1e-10), axis=1, keepdims=True)
    perp_ref[...] = jnp.exp(ent)


def _scalars_call(best_d, counts):
    return pl.pallas_call(
        _scalars_body,
        out_shape=[
            jax.ShapeDtypeStruct((1, 1), jnp.float32),
            jax.ShapeDtypeStruct((1, 1), jnp.float32),
        ],
    )(best_d, counts)


def kernel(inputs, weight):
    x = jnp.transpose(inputs, (0, 2, 3, 1))
    flat = x.reshape(-1, DIM)
    a = jnp.sum(flat ** 2, axis=1, keepdims=True)

    idx, best_d = _argmin_call(flat, a, weight)

    idx2d = idx.reshape(N_TOK // 128, 128)
    weight128 = jnp.pad(weight, ((0, 0), (0, 128 - DIM)))
    zeros = jnp.zeros((N_CODES,), jnp.float32)
    ones = jnp.ones((128,), jnp.float32)
    q3, counts = _sc_gather_hist(idx2d, weight128, zeros, ones)

    loss, perp = _scalars_call(best_d, counts)

    q_flat = q3.reshape(N_TOK, 128)[:, :DIM]
    quantized = jnp.transpose(q_flat.reshape(8, 32, 32, DIM), (0, 3, 1, 2))
    return quantized, loss[0, 0], perp[0, 0]


# speedup vs baseline: 1.0210x; 1.0210x over previous
import functools

import jax
import jax.numpy as jnp
from jax import lax
from jax.experimental import pallas as pl
from jax.experimental.pallas import tpu as pltpu
from jax.experimental.pallas import tpu_sc as plsc

N_CODES = 8192
DIM = 32
N_TOK = 8192
TOK_BLK = 2048
CHUNK = 4096
N_ELEMS = float(N_TOK * DIM)


def _argmin_body(x_ref, a_ref, w_ref, idx_ref, d_ref):
    x = (x_ref[...] * -2.0).astype(jnp.bfloat16).astype(jnp.float32)
    a = a_ref[...]
    col = lax.broadcasted_iota(jnp.int32, (TOK_BLK, CHUNK), 1).astype(jnp.float32)

    def step(k, carry):
        bd, bi = carry
        w = w_ref[pl.ds(k * CHUNK, CHUNK), :]
        mm = lax.dot_general(x, w, (((1,), (1,)), ((), ())),
                             preferred_element_type=jnp.float32)
        d = a + mm
        dmin = jnp.min(d, axis=1, keepdims=True)
        cand = jnp.where(d == dmin, col, jnp.float32(2**24))
        amin = jnp.min(cand, axis=1, keepdims=True).astype(jnp.int32) + k * CHUNK
        upd = dmin < bd
        return jnp.where(upd, dmin, bd), jnp.where(upd, amin, bi)

    half = N_CODES // 2 // CHUNK
    bd0 = jnp.full((TOK_BLK, 1), jnp.inf, jnp.float32)
    bi0 = jnp.zeros((TOK_BLK, 1), jnp.int32)
    bd, bi = lax.fori_loop(0, half, step, (bd0, bi0))
    bd = bd.astype(jnp.bfloat16).astype(jnp.float32)
    bd, bi = lax.fori_loop(half, 2 * half, step, (bd, bi))
    idx_ref[...] = bi
    d_ref[...] = bd


def _argmin_call(flat, a, weight):
    return pl.pallas_call(
        _argmin_body,
        grid=(N_TOK // TOK_BLK,),
        in_specs=[
            pl.BlockSpec((TOK_BLK, DIM), lambda i: (i, 0)),
            pl.BlockSpec((TOK_BLK, 1), lambda i: (i, 0)),
            pl.BlockSpec((N_CODES, DIM), lambda i: (0, 0)),
        ],
        out_specs=[
            pl.BlockSpec((TOK_BLK, 1), lambda i: (i, 0)),
            pl.BlockSpec((TOK_BLK, 1), lambda i: (i, 0)),
        ],
        out_shape=[
            jax.ShapeDtypeStruct((N_TOK, 1), jnp.int32),
            jax.ShapeDtypeStruct((N_TOK, 1), jnp.float32),
        ],
    )(flat, a, weight)


def _sc_gather_hist(idx2d, weight128, zeros, ones):
    mesh = plsc.VectorSubcoreMesh(core_axis_name="c", subcore_axis_name="s")
    nc, ns = mesh.num_cores, mesh.num_subcores
    nw = nc * ns
    rows_per_w = N_TOK // 128 // nw

    @functools.partial(
        pl.kernel,
        out_type=[
            jax.ShapeDtypeStruct((N_TOK // 128, 128, 128), jnp.float32),
            jax.ShapeDtypeStruct((nc, N_CODES), jnp.float32),
        ],
        mesh=mesh,
        scratch_types=[
            pltpu.VMEM((rows_per_w, 128), jnp.int32),
            pltpu.VMEM((rows_per_w, 128, 128), jnp.float32),
            pltpu.VMEM((128,), jnp.float32),
            pltpu.VMEM_SHARED((N_CODES,), jnp.float32),
            pltpu.SemaphoreType.DMA,
        ],
    )
    def k(idx_hbm, w_hbm, z_hbm, one_hbm, q_hbm, cnt_hbm,
          idx_v, rows_v, ones_v, shared, sem):
        cid = lax.axis_index("c")
        sid = lax.axis_index("s")
        wid = sid * nc + cid
        base = wid * rows_per_w

        pltpu.sync_copy(idx_hbm.at[pl.ds(base, rows_per_w)], idx_v)
        copies = [
            pltpu.async_copy(w_hbm.at[idx_v.at[j]], rows_v.at[j], sem)
            for j in range(rows_per_w)
        ]
        @pl.when(sid == 0)
        def _():
            pltpu.sync_copy(z_hbm, shared)
        pltpu.sync_copy(one_hbm, ones_v)
        for c in copies:
            c.wait()
        pltpu.sync_copy(rows_v, q_hbm.at[pl.ds(base, rows_per_w)])

        plsc.subcore_barrier()
        for j in range(rows_per_w):
            pltpu.sync_copy(ones_v, shared.at[idx_v.at[j]], add=True)
        plsc.subcore_barrier()

        @pl.when(sid == 0)
        def _():
            pltpu.sync_copy(shared, cnt_hbm.at[cid])

    return k(idx2d, weight128, zeros, ones)


def _scalars_body(d_ref, c_ref, loss_ref, perp_ref):
    sumd = jnp.sum(d_ref[...], keepdims=True)
    loss_ref[...] = sumd * (1.25 / N_ELEMS)
    p = jnp.sum(c_ref[...], axis=0, keepdims=True) * (1.0 / N_TOK)
    ent = -jnp.sum(p * jnp.log(p + 1e-10), axis=1, keepdims=True)
    perp_ref[...] = jnp.exp(ent)


def _scalars_call(best_d, counts):
    return pl.pallas_call(
        _scalars_body,
        out_shape=[
            jax.ShapeDtypeStruct((1, 1), jnp.float32),
            jax.ShapeDtypeStruct((1, 1), jnp.float32),
        ],
    )(best_d, counts)


def kernel(inputs, weight):
    x = jnp.transpose(inputs, (0, 2, 3, 1))
    flat = x.reshape(-1, DIM)
    a = jnp.sum(flat ** 2, axis=1, keepdims=True)

    idx, best_d = _argmin_call(flat, a, weight)

    idx2d = idx.reshape(N_TOK // 128, 128)
    weight128 = jnp.pad(weight, ((0, 0), (0, 128 - DIM)))
    zeros = jnp.zeros((N_CODES,), jnp.float32)
    ones = jnp.ones((128,), jnp.float32)
    q3, counts = _sc_gather_hist(idx2d, weight128, zeros, ones)

    loss, perp = _scalars_call(best_d, counts)

    q_flat = q3.reshape(N_TOK, 128)[:, :DIM]
    quantized = jnp.transpose(q_flat.reshape(8, 32, 32, DIM), (0, 3, 1, 2))
    return quantized, loss[0, 0], perp[0, 0]
